# baseline (speedup 1.0000x reference)
import functools

import jax
import jax.numpy as jnp
from jax import lax
from jax.experimental import pallas as pl
from jax.experimental.pallas import tpu as pltpu
from jax.experimental.pallas import tpu_sc as plsc

N_NODES = 10000
D = 128
N_WORKERS = 32
BLOCK = 320
NPAD = N_WORKERS * BLOCK
E_EDGES = 320000
LANES = 16
NFEAT = D // LANES

CHUNK = 2000
UNROLL = 5
RING = 4096
FB = 1024
LCAP = E_EDGES + FB

G = 64
LB = 2048
W_SUB = LB // G
SENTINEL = BLOCK


def _lin_relu_body(x_ref, w_ref, b_ref, o_ref):
    acc = jnp.dot(x_ref[...], w_ref[...], preferred_element_type=jnp.float32)
    o_ref[...] = jnp.maximum(acc + b_ref[...], 0.0)


def _tc_lin_relu(x, w_t, b):
    return pl.pallas_call(
        _lin_relu_body,
        out_shape=jax.ShapeDtypeStruct((x.shape[0], w_t.shape[1]), jnp.float32),
    )(x, w_t, b.reshape(1, -1))


def _upd_fused_body(x_ref, a_ref, wx_ref, wa_ref, b_ref, wp_ref, bp_ref,
                    h_ref, y_ref):
    acc = jnp.dot(x_ref[...], wx_ref[...], preferred_element_type=jnp.float32)
    acc += jnp.dot(a_ref[...], wa_ref[...], preferred_element_type=jnp.float32)
    h = jnp.maximum(acc + b_ref[...], 0.0)
    h_ref[...] = h
    y_ref[...] = jnp.maximum(
        jnp.dot(h, wp_ref[...], preferred_element_type=jnp.float32)
        + bp_ref[...], 0.0)


def _tc_upd_fused(x, agg, wx_t, wa_t, b, wp_t, bp):
    n = x.shape[0]
    return pl.pallas_call(
        _upd_fused_body,
        out_shape=(jax.ShapeDtypeStruct((n, D), jnp.float32),
                   jax.ShapeDtypeStruct((n, D), jnp.float32)),
    )(x, agg, wx_t, wa_t, b.reshape(1, -1), wp_t, bp.reshape(1, -1))


def _upd_body(x_ref, a_ref, wx_ref, wa_ref, b_ref, o_ref):
    acc = jnp.dot(x_ref[...], wx_ref[...], preferred_element_type=jnp.float32)
    acc += jnp.dot(a_ref[...], wa_ref[...], preferred_element_type=jnp.float32)
    o_ref[...] = jnp.maximum(acc + b_ref[...], 0.0)


def _tc_upd(x, agg, wx_t, wa_t, b):
    n = x.shape[0]
    return pl.pallas_call(
        _upd_body,
        out_shape=jax.ShapeDtypeStruct((n, D), jnp.float32),
    )(x, agg, wx_t, wa_t, b.reshape(1, -1))


def _sc_bucket_body(src_hbm, dst_hbm, lists_hbm, counts_hbm,
                    src_b0, src_b1, dst_b0, dst_b1, ring_v, cnt_v,
                    sem_s, sem_d):
    src_bufs = (src_b0, src_b1)
    dst_bufs = (dst_b0, dst_b1)
    wid = lax.axis_index("s") * 2 + lax.axis_index("c")
    lo = wid * BLOCK
    hi = lo + BLOCK
    n_chunks = E_EDGES // CHUNK

    def fire(g, p):
        goff = pl.multiple_of(g * CHUNK, 8)
        pltpu.async_copy(src_hbm.at[pl.ds(goff, CHUNK)],
                         src_bufs[p], sem_s)
        pltpu.async_copy(dst_hbm.at[pl.ds(goff, CHUNK)],
                         dst_bufs[p], sem_d)

    def wait(p):
        pltpu.make_async_copy(src_hbm.at[pl.ds(0, CHUNK)],
                              src_bufs[p], sem_s).wait()
        pltpu.make_async_copy(dst_hbm.at[pl.ds(0, CHUNK)],
                              dst_bufs[p], sem_d).wait()

    fire(0, 0)

    def do_pair(i, carry):
        tot, flushed = carry
        for p in (0, 1):
            g = 2 * i + p
            wait(p)

            @pl.when(g + 1 < n_chunks)
            def _():
                fire(g + 1, 1 - p)

            def scan(it, tot_):
                for u in range(UNROLL):
                    v = it * UNROLL + u
                    d = dst_bufs[p][pl.ds(v * LANES, LANES)]
                    s = src_bufs[p][pl.ds(v * LANES, LANES)]
                    m = (d >= lo) & (d < hi)
                    cum = plsc.cumsum(jnp.where(m, 1, 0))
                    pos = tot_ + cum - 1
                    ridx = pos & (RING - 1)
                    packed = (s << 9) + (d - lo)
                    plsc.store_scatter(ring_v, [ridx], packed, mask=m)
                    tot_ = tot_ + cum[LANES - 1]
                return tot_
            tot = lax.fori_loop(0, (CHUNK // LANES) // UNROLL, scan, tot)

            def flush_cond(c):
                return c[0] - c[1] >= FB

            def flush_body(c):
                t, f = c
                roff = pl.multiple_of(f & (RING - 1), 128)
                loff = pl.multiple_of(wid * LCAP + f, 128)
                pltpu.sync_copy(ring_v.at[pl.ds(roff, FB)],
                                lists_hbm.at[pl.ds(loff, FB)])
                return (t, f + FB)
            tot, flushed = lax.while_loop(flush_cond, flush_body,
                                          (tot, flushed))
        return (tot, flushed)

    tot, flushed = lax.fori_loop(0, n_chunks // 2, do_pair, (0, 0))

    sent = jnp.full((LANES,), SENTINEL, jnp.int32)
    iot = lax.iota(jnp.int32, LANES)
    for q in range(G // LANES):
        ridx = (tot + q * LANES + iot) & (RING - 1)
        plsc.store_scatter(ring_v, [ridx], sent,
                           mask=jnp.full((LANES,), True, jnp.bool_))

    def drain_cond(c):
        return c[1] < c[0] + G

    def drain_body(c):
        t, f = c
        roff = pl.multiple_of(f & (RING - 1), 128)
        loff = pl.multiple_of(wid * LCAP + f, 128)
        pltpu.sync_copy(ring_v.at[pl.ds(roff, FB)],
                        lists_hbm.at[pl.ds(loff, FB)])
        return (t, f + FB)
    _, _ = lax.while_loop(drain_cond, drain_body, (tot, flushed))

    for q in range(128 // LANES):
        cnt_v[pl.ds(q * LANES, LANES)] = jnp.full((LANES,), 0, jnp.int32) + tot
    pltpu.sync_copy(cnt_v, counts_hbm.at[pl.ds(pl.multiple_of(wid * 128, 128), 128)])


@functools.partial(
    pl.kernel,
    out_type=(jax.ShapeDtypeStruct((N_WORKERS * LCAP,), jnp.int32),
              jax.ShapeDtypeStruct((N_WORKERS * 128,), jnp.int32)),
    mesh=plsc.VectorSubcoreMesh(core_axis_name="c", subcore_axis_name="s"),
    compiler_params=pltpu.CompilerParams(needs_layout_passes=False),
    scratch_types=[
        pltpu.VMEM((CHUNK,), jnp.int32),
        pltpu.VMEM((CHUNK,), jnp.int32),
        pltpu.VMEM((CHUNK,), jnp.int32),
        pltpu.VMEM((CHUNK,), jnp.int32),
        pltpu.VMEM((RING,), jnp.int32),
        pltpu.VMEM((128,), jnp.int32),
        pltpu.SemaphoreType.DMA,
        pltpu.SemaphoreType.DMA,
    ],
)
def _sc_bucket(src_hbm, dst_hbm, lists_hbm, counts_hbm,
               src_b0, src_b1, dst_b0, dst_b1, ring_v, cnt_v, sem_s, sem_d):
    _sc_bucket_body(src_hbm, dst_hbm, lists_hbm, counts_hbm,
                    src_b0, src_b1, dst_b0, dst_b1, ring_v, cnt_v,
                    sem_s, sem_d)


DEPTH = 4


def _sc_gather_max_body(y_hbm, lists_hbm, counts_hbm, out_hbm,
                        agg_v, list_v, idx_v0, idx_v1, idx_v2, idx_v3,
                        rows_v0, rows_v1, rows_v2, rows_v3,
                        cnt_v, gsem0, gsem1, gsem2, gsem3):
    idx_bufs = (idx_v0, idx_v1, idx_v2, idx_v3)
    rows_bufs = (rows_v0, rows_v1, rows_v2, rows_v3)
    gsems = (gsem0, gsem1, gsem2, gsem3)
    wid = lax.axis_index("s") * 2 + lax.axis_index("c")
    lo = wid * BLOCK
    zeros = jnp.zeros((LANES,), jnp.float32)

    pltpu.sync_copy(counts_hbm.at[pl.ds(pl.multiple_of(wid * 128, 128), 128)], cnt_v)

    def init_row(r, _):
        for f in range(NFEAT):
            agg_v[r, pl.ds(f * LANES, LANES)] = zeros
        return 0
    lax.fori_loop(0, BLOCK + 1, init_row, 0)

    cntv = cnt_v[pl.ds(0, LANES)]
    cnt = cntv[0]
    n_blocks = (cnt + LB - 1) // LB

    def unpack_fire(w, p):
        for j in range(G // LANES):
            pk = list_v[pl.ds(w * G + j * LANES, LANES)]
            idx_bufs[p][pl.ds(j * LANES, LANES)] = pk >> 9
        pltpu.async_copy(y_hbm.at[idx_bufs[p]], rows_bufs[p], gsems[p])

    def wait_rows(p):
        pltpu.make_async_copy(y_hbm.at[idx_bufs[p]], rows_bufs[p],
                              gsems[p]).wait()

    def process(w, p):
        for j in range(G // LANES):
            dlv = list_v[pl.ds(w * G + j * LANES, LANES)] & 511
            for r in range(LANES):
                dl = dlv[r]
                row = j * LANES + r
                for f in range(NFEAT):
                    sl = pl.ds(f * LANES, LANES)
                    agg_v[dl, sl] = jnp.maximum(agg_v[dl, sl],
                                                rows_bufs[p][row, sl])

    def do_block(b, _):
        pltpu.sync_copy(
            lists_hbm.at[pl.ds(pl.multiple_of(wid * LCAP + b * LB, 8), LB)],
            list_v)
        rem = cnt - b * LB
        wsub = jnp.minimum(W_SUB, (rem + G - 1) // G)
        for p in range(DEPTH):
            @pl.when(p < wsub)
            def _(p=p):
                unpack_fire(p, p)

        def grp(q, _):
            for p in range(DEPTH):
                k = DEPTH * q + p

                @pl.when(k < wsub)
                def _(k=k, p=p):
                    wait_rows(p)
                    process(k, p)

                    @pl.when(k + DEPTH < wsub)
                    def _():
                        unpack_fire(k + DEPTH, p)
            return 0
        lax.fori_loop(0, (wsub + DEPTH - 1) // DEPTH, grp, 0)
        return 0

    lax.fori_loop(0, n_blocks, do_block, 0)

    pltpu.sync_copy(agg_v.at[pl.ds(0, BLOCK)], out_hbm.at[pl.ds(lo, BLOCK)])


@functools.partial(
    pl.kernel,
    out_type=jax.ShapeDtypeStruct((NPAD, D), jnp.float32),
    mesh=plsc.VectorSubcoreMesh(core_axis_name="c", subcore_axis_name="s"),
    compiler_params=pltpu.CompilerParams(needs_layout_passes=False),
    scratch_types=[
        pltpu.VMEM((BLOCK + 1, D), jnp.float32),
        pltpu.VMEM((LB,), jnp.int32),
        pltpu.VMEM((G,), jnp.int32),
        pltpu.VMEM((G,), jnp.int32),
        pltpu.VMEM((G,), jnp.int32),
        pltpu.VMEM((G,), jnp.int32),
        pltpu.VMEM((G, D), jnp.float32),
        pltpu.VMEM((G, D), jnp.float32),
        pltpu.VMEM((G, D), jnp.float32),
        pltpu.VMEM((G, D), jnp.float32),
        pltpu.VMEM((128,), jnp.int32),
        pltpu.SemaphoreType.DMA,
        pltpu.SemaphoreType.DMA,
        pltpu.SemaphoreType.DMA,
        pltpu.SemaphoreType.DMA,
    ],
)
def _sc_gather_max(y_hbm, lists_hbm, counts_hbm, out_hbm,
                   agg_v, list_v, idx_v0, idx_v1, idx_v2, idx_v3,
                   rows_v0, rows_v1, rows_v2, rows_v3,
                   cnt_v, gsem0, gsem1, gsem2, gsem3):
    _sc_gather_max_body(y_hbm, lists_hbm, counts_hbm, out_hbm,
                        agg_v, list_v, idx_v0, idx_v1, idx_v2, idx_v3,
                        rows_v0, rows_v1, rows_v2, rows_v3,
                        cnt_v, gsem0, gsem1, gsem2, gsem3)


def kernel(x, edge_index, W_pool1, b_pool1, W_upd1, b_upd1,
           W_pool2, b_pool2, W_upd2, b_upd2):
    ei = edge_index.astype(jnp.int32)
    src, dst = ei[0], ei[1]

    wp1_t = W_pool1.T
    wu1x_t = W_upd1[:, :D].T
    wu1a_t = W_upd1[:, D:].T
    wp2_t = W_pool2.T
    wu2x_t = W_upd2[:, :D].T
    wu2a_t = W_upd2[:, D:].T

    lists, counts = _sc_bucket(src, dst)
    y1 = _tc_lin_relu(x, wp1_t, b_pool1)
    agg1 = _sc_gather_max(y1, lists, counts)[:N_NODES]
    h1, y2 = _tc_upd_fused(x, agg1, wu1x_t, wu1a_t, b_upd1, wp2_t, b_pool2)
    agg2 = _sc_gather_max(y2, lists, counts)[:N_NODES]
    return _tc_upd(h1, agg2, wu2x_t, wu2a_t, b_upd2)


# speedup vs baseline: 1.5934x; 1.5934x over previous
import functools

import jax
import jax.numpy as jnp
from jax import lax
from jax.experimental import pallas as pl
from jax.experimental.pallas import tpu as pltpu
from jax.experimental.pallas import tpu_sc as plsc

N_NODES = 10000
D = 128
N_WORKERS = 32
BLOCK = 320
NPAD = N_WORKERS * BLOCK
E_EDGES = 320000
LANES = 16
NFEAT = D // LANES

CHUNK = 2000
UNROLL = 5
RING = 4096
FB = 1024
LCAP = E_EDGES + FB

G = 64
DW = D // 2
LB = 2048
W_SUB = LB // G
SENTINEL = BLOCK


def _lin_relu_body(x_ref, w_ref, b_ref, o_ref):
    acc = jnp.dot(x_ref[...], w_ref[...], preferred_element_type=jnp.float32)
    o_ref[...] = jnp.maximum(acc + b_ref[...], 0.0).astype(jnp.bfloat16)


def _tc_lin_relu(x, w_t, b):
    return pl.pallas_call(
        _lin_relu_body,
        out_shape=jax.ShapeDtypeStruct((x.shape[0], w_t.shape[1]),
                                       jnp.bfloat16),
    )(x, w_t, b.reshape(1, -1))


def _upd_fused_body(x_ref, a_ref, wx_ref, wa_ref, b_ref, wp_ref, bp_ref,
                    h_ref, y_ref):
    acc = jnp.dot(x_ref[...], wx_ref[...], preferred_element_type=jnp.float32)
    acc += jnp.dot(a_ref[...].astype(jnp.float32), wa_ref[...],
                   preferred_element_type=jnp.float32)
    h = jnp.maximum(acc + b_ref[...], 0.0)
    h_ref[...] = h
    y_ref[...] = jnp.maximum(
        jnp.dot(h, wp_ref[...], preferred_element_type=jnp.float32)
        + bp_ref[...], 0.0).astype(jnp.bfloat16)


def _tc_upd_fused(x, agg, wx_t, wa_t, b, wp_t, bp):
    n = x.shape[0]
    return pl.pallas_call(
        _upd_fused_body,
        out_shape=(jax.ShapeDtypeStruct((n, D), jnp.float32),
                   jax.ShapeDtypeStruct((n, D), jnp.bfloat16)),
    )(x, agg, wx_t, wa_t, b.reshape(1, -1), wp_t, bp.reshape(1, -1))


def _upd_body(x_ref, a_ref, wx_ref, wa_ref, b_ref, o_ref):
    acc = jnp.dot(x_ref[...], wx_ref[...], preferred_element_type=jnp.float32)
    acc += jnp.dot(a_ref[...].astype(jnp.float32), wa_ref[...],
                   preferred_element_type=jnp.float32)
    o_ref[...] = jnp.maximum(acc + b_ref[...], 0.0)


def _tc_upd(x, agg, wx_t, wa_t, b):
    n = x.shape[0]
    return pl.pallas_call(
        _upd_body,
        out_shape=jax.ShapeDtypeStruct((n, D), jnp.float32),
    )(x, agg, wx_t, wa_t, b.reshape(1, -1))


def _sc_bucket_body(src_hbm, dst_hbm, lists_hbm, counts_hbm,
                    src_b0, src_b1, dst_b0, dst_b1, ring_v, cnt_v,
                    sem_s, sem_d):
    src_bufs = (src_b0, src_b1)
    dst_bufs = (dst_b0, dst_b1)
    wid = lax.axis_index("s") * 2 + lax.axis_index("c")
    lo = wid * BLOCK
    hi = lo + BLOCK
    n_chunks = E_EDGES // CHUNK

    def fire(g, p):
        goff = pl.multiple_of(g * CHUNK, 8)
        pltpu.async_copy(src_hbm.at[pl.ds(goff, CHUNK)],
                         src_bufs[p], sem_s)
        pltpu.async_copy(dst_hbm.at[pl.ds(goff, CHUNK)],
                         dst_bufs[p], sem_d)

    def wait(p):
        pltpu.make_async_copy(src_hbm.at[pl.ds(0, CHUNK)],
                              src_bufs[p], sem_s).wait()
        pltpu.make_async_copy(dst_hbm.at[pl.ds(0, CHUNK)],
                              dst_bufs[p], sem_d).wait()

    fire(0, 0)

    def do_pair(i, carry):
        tot, flushed = carry
        for p in (0, 1):
            g = 2 * i + p
            wait(p)

            @pl.when(g + 1 < n_chunks)
            def _():
                fire(g + 1, 1 - p)

            def scan(it, tot_):
                for u in range(UNROLL):
                    v = it * UNROLL + u
                    d = dst_bufs[p][pl.ds(v * LANES, LANES)]
                    s = src_bufs[p][pl.ds(v * LANES, LANES)]
                    m = (d >= lo) & (d < hi)
                    cum = plsc.cumsum(jnp.where(m, 1, 0))
                    pos = tot_ + cum - 1
                    ridx = pos & (RING - 1)
                    packed = (s << 9) + (d - lo)
                    plsc.store_scatter(ring_v, [ridx], packed, mask=m)
                    tot_ = tot_ + cum[LANES - 1]
                return tot_
            tot = lax.fori_loop(0, (CHUNK // LANES) // UNROLL, scan, tot)

            def flush_cond(c):
                return c[0] - c[1] >= FB

            def flush_body(c):
                t, f = c
                roff = pl.multiple_of(f & (RING - 1), 128)
                loff = pl.multiple_of(wid * LCAP + f, 128)
                pltpu.sync_copy(ring_v.at[pl.ds(roff, FB)],
                                lists_hbm.at[pl.ds(loff, FB)])
                return (t, f + FB)
            tot, flushed = lax.while_loop(flush_cond, flush_body,
                                          (tot, flushed))
        return (tot, flushed)

    tot, flushed = lax.fori_loop(0, n_chunks // 2, do_pair, (0, 0))

    sent = jnp.full((LANES,), SENTINEL, jnp.int32)
    iot = lax.iota(jnp.int32, LANES)
    for q in range(G // LANES):
        ridx = (tot + q * LANES + iot) & (RING - 1)
        plsc.store_scatter(ring_v, [ridx], sent,
                           mask=jnp.full((LANES,), True, jnp.bool_))

    def drain_cond(c):
        return c[1] < c[0] + G

    def drain_body(c):
        t, f = c
        roff = pl.multiple_of(f & (RING - 1), 128)
        loff = pl.multiple_of(wid * LCAP + f, 128)
        pltpu.sync_copy(ring_v.at[pl.ds(roff, FB)],
                        lists_hbm.at[pl.ds(loff, FB)])
        return (t, f + FB)
    _, _ = lax.while_loop(drain_cond, drain_body, (tot, flushed))

    for q in range(128 // LANES):
        cnt_v[pl.ds(q * LANES, LANES)] = jnp.full((LANES,), 0, jnp.int32) + tot
    pltpu.sync_copy(cnt_v, counts_hbm.at[pl.ds(pl.multiple_of(wid * 128, 128), 128)])


@functools.partial(
    pl.kernel,
    out_type=(jax.ShapeDtypeStruct((N_WORKERS * LCAP,), jnp.int32),
              jax.ShapeDtypeStruct((N_WORKERS * 128,), jnp.int32)),
    mesh=plsc.VectorSubcoreMesh(core_axis_name="c", subcore_axis_name="s"),
    compiler_params=pltpu.CompilerParams(needs_layout_passes=False),
    scratch_types=[
        pltpu.VMEM((CHUNK,), jnp.int32),
        pltpu.VMEM((CHUNK,), jnp.int32),
        pltpu.VMEM((CHUNK,), jnp.int32),
        pltpu.VMEM((CHUNK,), jnp.int32),
        pltpu.VMEM((RING,), jnp.int32),
        pltpu.VMEM((128,), jnp.int32),
        pltpu.SemaphoreType.DMA,
        pltpu.SemaphoreType.DMA,
    ],
)
def _sc_bucket(src_hbm, dst_hbm, lists_hbm, counts_hbm,
               src_b0, src_b1, dst_b0, dst_b1, ring_v, cnt_v, sem_s, sem_d):
    _sc_bucket_body(src_hbm, dst_hbm, lists_hbm, counts_hbm,
                    src_b0, src_b1, dst_b0, dst_b1, ring_v, cnt_v,
                    sem_s, sem_d)


DEPTH = 2


def _sc_gather_max_body(y_hbm, lists_hbm, counts_hbm, out_hbm,
                        ysh, agg_v, list_v, idx_v0, idx_v1, idx_v2, idx_v3,
                        rows_v0, rows_v1, rows_v2, rows_v3,
                        cnt_v, gsem0, gsem1, gsem2, gsem3):
    idx_bufs = (idx_v0, idx_v1, idx_v2, idx_v3)
    rows_bufs = (rows_v0, rows_v1, rows_v2, rows_v3)
    gsems = (gsem0, gsem1, gsem2, gsem3)
    wid = lax.axis_index("s") * 2 + lax.axis_index("c")
    lo = wid * BLOCK
    zeros = jnp.zeros((LANES,), jnp.int32)

    @pl.when(lax.axis_index("s") == 0)
    def _():
        pltpu.sync_copy(y_hbm, ysh)

    pltpu.sync_copy(counts_hbm.at[pl.ds(pl.multiple_of(wid * 128, 128), 128)], cnt_v)

    def init_row(r, _):
        for f in range(DW // LANES):
            agg_v[r, pl.ds(f * LANES, LANES)] = zeros
        return 0
    lax.fori_loop(0, BLOCK + 1, init_row, 0)
    plsc.subcore_barrier()

    cntv = cnt_v[pl.ds(0, LANES)]
    cnt = cntv[0]
    n_blocks = (cnt + LB - 1) // LB

    def unpack_fire(w, p):
        for j in range(G // LANES):
            pk = list_v[pl.ds(w * G + j * LANES, LANES)]
            idx_bufs[p][pl.ds(j * LANES, LANES)] = pk >> 9
        pltpu.async_copy(ysh.at[idx_bufs[p]], rows_bufs[p], gsems[p])

    def wait_rows(p):
        pltpu.make_async_copy(ysh.at[idx_bufs[p]], rows_bufs[p],
                              gsems[p]).wait()

    def process(w, p):
        for j in range(G // LANES):
            dlv = list_v[pl.ds(w * G + j * LANES, LANES)] & 511
            for r in range(LANES):
                dl = dlv[r]
                row = j * LANES + r
                for f in range(DW // LANES):
                    sl = pl.ds(f * LANES, LANES)
                    a = plsc.bitcast(agg_v[dl, sl], jnp.bfloat16)
                    v = plsc.bitcast(rows_bufs[p][row, sl], jnp.bfloat16)
                    agg_v[dl, sl] = plsc.bitcast(jnp.maximum(a, v),
                                                 jnp.int32)

    def do_block(b, _):
        pltpu.sync_copy(
            lists_hbm.at[pl.ds(pl.multiple_of(wid * LCAP + b * LB, 8), LB)],
            list_v)
        rem = cnt - b * LB
        wsub = jnp.minimum(W_SUB, (rem + G - 1) // G)
        for p in range(DEPTH):
            @pl.when(p < wsub)
            def _(p=p):
                unpack_fire(p, p)

        def grp(q, _):
            for p in range(DEPTH):
                k = DEPTH * q + p

                @pl.when(k < wsub)
                def _(k=k, p=p):
                    wait_rows(p)
                    process(k, p)

                    @pl.when(k + DEPTH < wsub)
                    def _():
                        unpack_fire(k + DEPTH, p)
            return 0
        lax.fori_loop(0, (wsub + DEPTH - 1) // DEPTH, grp, 0)
        return 0

    lax.fori_loop(0, n_blocks, do_block, 0)

    pltpu.sync_copy(agg_v.at[pl.ds(0, BLOCK)], out_hbm.at[pl.ds(lo, BLOCK)])


@functools.partial(
    pl.kernel,
    out_type=jax.ShapeDtypeStruct((NPAD, DW), jnp.int32),
    mesh=plsc.VectorSubcoreMesh(core_axis_name="c", subcore_axis_name="s"),
    compiler_params=pltpu.CompilerParams(needs_layout_passes=False),
    scratch_types=[
        pltpu.VMEM_SHARED((N_NODES, DW), jnp.int32),
        pltpu.VMEM((BLOCK + 1, DW), jnp.int32),
        pltpu.VMEM((LB,), jnp.int32),
        pltpu.VMEM((G,), jnp.int32),
        pltpu.VMEM((G,), jnp.int32),
        pltpu.VMEM((G,), jnp.int32),
        pltpu.VMEM((G,), jnp.int32),
        pltpu.VMEM((G, DW), jnp.int32),
        pltpu.VMEM((G, DW), jnp.int32),
        pltpu.VMEM((G, DW), jnp.int32),
        pltpu.VMEM((G, DW), jnp.int32),
        pltpu.VMEM((128,), jnp.int32),
        pltpu.SemaphoreType.DMA,
        pltpu.SemaphoreType.DMA,
        pltpu.SemaphoreType.DMA,
        pltpu.SemaphoreType.DMA,
    ],
)
def _sc_gather_max(y_hbm, lists_hbm, counts_hbm, out_hbm,
                   ysh, agg_v, list_v, idx_v0, idx_v1, idx_v2, idx_v3,
                   rows_v0, rows_v1, rows_v2, rows_v3,
                   cnt_v, gsem0, gsem1, gsem2, gsem3):
    _sc_gather_max_body(y_hbm, lists_hbm, counts_hbm, out_hbm,
                        ysh, agg_v, list_v, idx_v0, idx_v1, idx_v2, idx_v3,
                        rows_v0, rows_v1, rows_v2, rows_v3,
                        cnt_v, gsem0, gsem1, gsem2, gsem3)


def kernel(x, edge_index, W_pool1, b_pool1, W_upd1, b_upd1,
           W_pool2, b_pool2, W_upd2, b_upd2):
    ei = edge_index.astype(jnp.int32)
    src, dst = ei[0], ei[1]

    wp1_t = W_pool1.T
    wu1x_t = W_upd1[:, :D].T
    wu1a_t = W_upd1[:, D:].T
    wp2_t = W_pool2.T
    wu2x_t = W_upd2[:, :D].T
    wu2a_t = W_upd2[:, D:].T

    def pack(y):
        return jax.lax.bitcast_convert_type(y.reshape(-1, DW, 2), jnp.int32)

    def unpack(a):
        return jax.lax.bitcast_convert_type(
            a, jnp.bfloat16).reshape(-1, D)[:N_NODES]

    lists, counts = _sc_bucket(src, dst)
    y1 = _tc_lin_relu(x, wp1_t, b_pool1)
    agg1 = unpack(_sc_gather_max(pack(y1), lists, counts))
    h1, y2 = _tc_upd_fused(x, agg1, wu1x_t, wu1a_t, b_upd1, wp2_t, b_pool2)
    agg2 = unpack(_sc_gather_max(pack(y2), lists, counts))
    return _tc_upd(h1, agg2, wu2x_t, wu2a_t, b_upd2)
